# Initial kernel scaffold; baseline (speedup 1.0000x reference)
import jax
import jax.numpy as jnp
from jax.experimental import pallas as pl


def kernel(input_tensor, training_data_features, training_data_labels):
    raise NotImplementedError("write your pallas kernel here")


# speedup vs baseline: 3.2740x; 3.2740x over previous
import functools

import jax
import jax.numpy as jnp
from jax import lax
from jax.experimental import pallas as pl

N_TRAIN = 100000
FEAT = 27
N_APP = 10
K = 3
BIGF = 3.0e38


def _knn_body(q_ref, xt_ref, dist_ref, idx_ref):
    xt = xt_ref[...]
    q = q_ref[...]
    scale = jnp.max(jnp.abs(xt), axis=1, keepdims=True)
    inv = jnp.where(scale != 0.0, 1.0 / scale, 0.0)
    qs = q * inv
    diff = xt * inv - qs
    d2 = jnp.sum(diff * diff, axis=0, keepdims=True)

    iota = lax.broadcasted_iota(jnp.int32, (1, N_TRAIN), 1)
    big_i = jnp.int32(N_TRAIN)
    cur = d2
    ms, idxs = [], []
    for _ in range(K):
        m = jnp.min(cur)
        i = jnp.min(jnp.where(cur == m, iota, big_i))
        ms.append(m)
        idxs.append(i)
        cur = jnp.where(iota == i, BIGF, cur)

    ds = [jnp.sqrt(m) for m in ms]
    li = lax.broadcasted_iota(jnp.int32, (1, 8), 1)
    dist_ref[...] = jnp.where(
        li == 0, ds[0], jnp.where(li == 1, ds[1], jnp.where(li == 2, ds[2], 0.0))
    )
    idx_ref[...] = jnp.where(
        li == 0, idxs[0], jnp.where(li == 1, idxs[1], jnp.where(li == 2, idxs[2], 0))
    )


def _knn_tc(input_tensor, feats_t, interpret=False):
    dist, idx = pl.pallas_call(
        _knn_body,
        out_shape=(
            jax.ShapeDtypeStruct((1, 8), jnp.float32),
            jax.ShapeDtypeStruct((1, 8), jnp.int32),
        ),
        interpret=interpret,
    )(input_tensor, feats_t)
    return dist, idx


def kernel(input_tensor, training_data_features, training_data_labels):
    feats_t = training_data_features.T
    dist, idx = _knn_tc(input_tensor, feats_t)
    k_dist = dist[0, :K]
    k_idx = idx[0, :K]
    rows = jnp.take(training_data_labels, k_idx, axis=0)
    d_safe = jnp.where(k_dist == 0.0, 1.0, k_dist)
    w = 1.0 / d_safe
    weighted = jnp.sum(rows * w[:, None], axis=0) / jnp.sum(w)
    result = jnp.where(k_dist[0] == 0.0, rows[0], weighted)
    return k_dist, result


# speedup vs baseline: 9.3707x; 2.8622x over previous
import functools

import jax
import jax.numpy as jnp
from jax import lax
from jax.experimental import pallas as pl

N_TRAIN = 100000
FEAT = 27
N_APP = 10
K = 3
BIGF = 3.0e38


def _knn_body(q_ref, xt_ref, labt_ref, dist_ref, res_ref):
    xt = xt_ref[...]
    q = q_ref[...]
    scale = jnp.max(jnp.abs(xt), axis=1, keepdims=True)
    inv = jnp.where(scale != 0.0, 1.0 / scale, 0.0)
    qs = q * inv
    diff = xt * inv - qs
    d2 = jnp.sum(diff * diff, axis=0, keepdims=True)

    iota = lax.broadcasted_iota(jnp.int32, (1, N_TRAIN), 1)
    big_i = jnp.int32(N_TRAIN)
    cur = d2
    ms, idxs = [], []
    for _ in range(K):
        m = jnp.min(cur)
        i = jnp.min(jnp.where(cur == m, iota, big_i))
        ms.append(m)
        idxs.append(i)
        cur = jnp.where(iota == i, BIGF, cur)

    ds = [jnp.sqrt(m) for m in ms]
    dsafe = [jnp.where(d == 0.0, 1.0, d) for d in ds]
    w = [1.0 / d for d in dsafe]
    denom = w[0] + w[1] + w[2]
    exact = ds[0] == 0.0
    a = [jnp.where(exact, 1.0 if k == 0 else 0.0, w[k] / denom) for k in range(K)]
    w_sparse = (
        jnp.where(iota == idxs[0], a[0], 0.0)
        + jnp.where(iota == idxs[1], a[1], 0.0)
        + jnp.where(iota == idxs[2], a[2], 0.0)
    )
    labt = labt_ref[...]
    res_ref[...] = lax.dot_general(
        labt, w_sparse,
        dimension_numbers=(((1,), (1,)), ((), ())),
        preferred_element_type=jnp.float32,
    )

    li = lax.broadcasted_iota(jnp.int32, (1, 8), 1)
    dist_ref[...] = jnp.where(
        li == 0, ds[0], jnp.where(li == 1, ds[1], jnp.where(li == 2, ds[2], 0.0))
    )


def _knn_tc(input_tensor, feats_t, labels_t, interpret=False):
    dist, res = pl.pallas_call(
        _knn_body,
        out_shape=(
            jax.ShapeDtypeStruct((1, 8), jnp.float32),
            jax.ShapeDtypeStruct((N_APP, 1), jnp.float32),
        ),
        interpret=interpret,
    )(input_tensor, feats_t, labels_t)
    return dist, res


def kernel(input_tensor, training_data_features, training_data_labels):
    feats_t = training_data_features.T
    labels_t = training_data_labels.T
    dist, res = _knn_tc(input_tensor, feats_t, labels_t)
    return dist[0, :K], res[:, 0]
